# baseline (speedup 1.0000x reference)
import math
from functools import partial

import jax
import jax.numpy as jnp
from jax.experimental import pallas as pl
from jax.experimental.pallas import tpu as pltpu


def _conv1_kernel(x_ref, wq_ref, b_ref, o_ref, acc_ref, *, bgrp):
    x = x_ref[...]
    m = bgrp * 32 * 32
    pieces = []
    for dy in range(3):
        for dx in range(3):
            r = x
            if dy:
                r = jnp.concatenate([r[:, dy:], r[:, :dy]], axis=1)
            if dx:
                r = jnp.concatenate([r[:, :, dx:], r[:, :, :dx]], axis=2)
            pieces.append(r)
    p = jnp.concatenate(pieces, axis=-1).reshape(m, 216)
    acc_ref[...] = jnp.dot(p, wq_ref[...],
                           preferred_element_type=jnp.float32)
    bias = b_ref[...]
    a = jnp.maximum(acc_ref[...].reshape(bgrp, 16, 2, 32, 512) + bias, 0.0)
    hmax = jnp.maximum(a[:, :, 0], a[:, :, 1])
    hr = hmax.reshape(bgrp, 16, 16, 1024)
    pooled = jnp.maximum(hr[..., 0:512], hr[..., 512:1024])
    hp = jax.lax.broadcasted_iota(jnp.int32, pooled.shape, 1)
    wp = jax.lax.broadcasted_iota(jnp.int32, pooled.shape, 2)
    interior = ((hp >= 1) & (hp <= 14) & (wp >= 1) & (wp <= 14))
    o_ref[...] = jnp.where(interior, pooled, 0.0).astype(o_ref.dtype)


def _conv1(x, w1, b1, *, bgrp=4):
    n = x.shape[0]
    g = n // 8
    xg = x.reshape(g, 8, 3, 28, 28).transpose(0, 3, 4, 1, 2)
    xg = xg.reshape(g, 28, 28, 24).astype(jnp.bfloat16)
    xg = jnp.pad(xg, ((0, 0), (3, 1), (3, 1), (0, 0)))
    w9 = w1.reshape(9, 1, 3, 1, 64).astype(jnp.bfloat16)
    wq = (jnp.eye(8, dtype=jnp.bfloat16)[None, :, None, :, None]
          * w9).reshape(216, 512)
    bt = jnp.tile(b1, 8)
    yg = pl.pallas_call(
        partial(_conv1_kernel, bgrp=bgrp),
        out_shape=jax.ShapeDtypeStruct((g, 16, 16, 512), jnp.bfloat16),
        grid=(g // bgrp,),
        in_specs=[
            pl.BlockSpec((bgrp, 32, 32, 24), lambda i: (i, 0, 0, 0)),
            pl.BlockSpec((216, 512), lambda i: (0, 0)),
            pl.BlockSpec((1, 512), lambda i: (0, 0)),
        ],
        out_specs=pl.BlockSpec((bgrp, 16, 16, 512), lambda i: (i, 0, 0, 0)),
        scratch_shapes=[pltpu.VMEM((bgrp * 1024, 512), jnp.float32)],
        compiler_params=pltpu.CompilerParams(
            dimension_semantics=("parallel",)),
    )(xg, wq, bt.reshape(1, 512))
    return yg


def _conv2_kernel(x_ref, w_ref, b_ref, o_ref, p_ref, acc_ref, *, bg):
    x = x_ref[...]
    m1 = bg * 224
    for i in range(8):
        xi = x[..., i * 64:(i + 1) * 64]
        for dy in range(3):
            xs = xi[:, dy: dy + 14]
            for dx in range(3):
                t = dy * 3 + dx
                r = xs
                if dx:
                    r = jnp.concatenate([r[:, :, dx:], r[:, :, :dx]], axis=2)
                p_ref[pl.ds(i * m1, m1), pl.ds(t * 64, 64)] = (
                    r.reshape(m1, 64))
    acc_ref[...] = jnp.dot(p_ref[...], w_ref[...].reshape(576, 512),
                           preferred_element_type=jnp.float32)
    bias = b_ref[...]
    for i in range(8):
        a = acc_ref[pl.ds(i * m1, m1), :]
        z = jnp.maximum(a + bias, 0.0).reshape(bg, 7, 2, 16, 512)
        hs = z[:, :, 0] + z[:, :, 1]
        hr = hs.reshape(bg, 7, 8, 1024)
        pooled = (hr[..., 0:512] + hr[..., 512:1024]) * 0.25
        o_ref[i] = pooled.astype(o_ref.dtype)


def _conv2(yg, w2, b2, *, bg=2):
    g = yg.shape[0]
    wk = w2.reshape(9, 64, 512).astype(jnp.bfloat16)
    return pl.pallas_call(
        partial(_conv2_kernel, bg=bg),
        out_shape=jax.ShapeDtypeStruct((8, g, 7, 8, 512), jnp.bfloat16),
        grid=(g // bg,),
        in_specs=[
            pl.BlockSpec((bg, 16, 16, 512), lambda s: (s, 0, 0, 0)),
            pl.BlockSpec((9, 64, 512), lambda s: (0, 0, 0)),
            pl.BlockSpec((1, 512), lambda s: (0, 0)),
        ],
        out_specs=pl.BlockSpec((8, bg, 7, 8, 512), lambda s: (0, s, 0, 0, 0)),
        scratch_shapes=[pltpu.VMEM((8 * bg * 224, 576), jnp.bfloat16),
                        pltpu.VMEM((8 * bg * 224, 512), jnp.float32)],
        compiler_params=pltpu.CompilerParams(
            dimension_semantics=("parallel",)),
    )(yg, wk, b2.reshape(1, 512))


def _fc1_kernel(x_ref, w_ref, b_ref, o_ref, acc_ref):
    k = pl.program_id(1)

    @pl.when(k == 0)
    def _():
        acc_ref[...] = jnp.zeros_like(acc_ref)

    acc_ref[...] += jnp.dot(x_ref[...], w_ref[...],
                            preferred_element_type=jnp.float32)

    @pl.when(k == pl.num_programs(1) - 1)
    def _():
        o_ref[...] = jnp.maximum(acc_ref[...] + b_ref[...],
                                 0.0).astype(o_ref.dtype)


def _fc1(x, w, b, *, tn=2048, tk=3584):
    bsz, kdim = x.shape
    ndim = w.shape[1]
    return pl.pallas_call(
        _fc1_kernel,
        out_shape=jax.ShapeDtypeStruct((bsz, ndim), jnp.bfloat16),
        grid=(ndim // tn, kdim // tk),
        in_specs=[
            pl.BlockSpec((bsz, tk), lambda j, k: (0, k)),
            pl.BlockSpec((tk, tn), lambda j, k: (k, j)),
            pl.BlockSpec((1, tn), lambda j, k: (0, j)),
        ],
        out_specs=pl.BlockSpec((bsz, tn), lambda j, k: (0, j)),
        scratch_shapes=[pltpu.VMEM((bsz, tn), jnp.float32)],
        compiler_params=pltpu.CompilerParams(
            dimension_semantics=("parallel", "arbitrary"),
            vmem_limit_bytes=48 * 1024 * 1024),
        cost_estimate=pl.CostEstimate(
            flops=2 * bsz * kdim * ndim, transcendentals=0,
            bytes_accessed=kdim * ndim * 2 + bsz * kdim * 2 + bsz * ndim * 2),
    )(x, w, b.reshape(1, ndim))


def _fc23_kernel(x_ref, w2_ref, b2_ref, w3_ref, b3_ref, o_ref, h_ref):
    h = jnp.dot(x_ref[...], w2_ref[...], preferred_element_type=jnp.float32)
    h_ref[...] = jnp.maximum(h + b2_ref[...], 0.0).astype(jnp.bfloat16)
    y = jnp.dot(h_ref[...], w3_ref[...], preferred_element_type=jnp.float32)
    o_ref[...] = (y + b3_ref[...])[:, :10]


def _fc23(x, w2, b2, w3, b3, *, bm=128):
    bsz, kdim = x.shape
    n2 = w2.shape[1]
    return pl.pallas_call(
        _fc23_kernel,
        out_shape=jax.ShapeDtypeStruct((bsz, 10), jnp.float32),
        grid=(bsz // bm,),
        in_specs=[
            pl.BlockSpec((bm, kdim), lambda i: (i, 0)),
            pl.BlockSpec((kdim, n2), lambda i: (0, 0)),
            pl.BlockSpec((1, n2), lambda i: (0, 0)),
            pl.BlockSpec((n2, w3.shape[1]), lambda i: (0, 0)),
            pl.BlockSpec((1, w3.shape[1]), lambda i: (0, 0)),
        ],
        out_specs=pl.BlockSpec((bm, 10), lambda i: (i, 0)),
        scratch_shapes=[pltpu.VMEM((bm, n2), jnp.bfloat16)],
        compiler_params=pltpu.CompilerParams(
            dimension_semantics=("parallel",),
            vmem_limit_bytes=48 * 1024 * 1024),
        cost_estimate=pl.CostEstimate(
            flops=2 * bsz * kdim * (n2 + w3.shape[1]), transcendentals=0,
            bytes_accessed=kdim * n2 * 2 + bsz * kdim * 2),
    )(x, w2, b2.reshape(1, n2), w3, b3.reshape(1, w3.shape[1]))


def kernel(x, conv1_w, conv1_b, conv2_w, conv2_b,
           fc1_w, fc1_b, fc2_w, fc2_b, fc3_w, fc3_b):
    n = x.shape[0]
    g = n // 8
    yg = _conv1(x, conv1_w, conv1_b, bgrp=min(4, g))
    y2 = _conv2(yg, conv2_w, conv2_b, bg=min(2, g))
    flat = jnp.transpose(y2[:, :, :, :7, :],
                         (1, 0, 4, 2, 3)).reshape(n, 512 * 49)
    h1 = _fc1(flat, fc1_w, fc1_b)
    return _fc23(h1, fc2_w, fc2_b, fc3_w, fc3_b,
                 bm=min(128, n))


# speedup vs baseline: 1.0003x; 1.0003x over previous
import math
from functools import partial

import jax
import jax.numpy as jnp
from jax.experimental import pallas as pl
from jax.experimental.pallas import tpu as pltpu


def _conv1_kernel(x_ref, wq_ref, b_ref, o_ref, acc_ref, *, bgrp):
    x = x_ref[...]
    m = bgrp * 32 * 32
    pieces = []
    for dy in range(3):
        for dx in range(3):
            r = x
            if dy:
                r = jnp.concatenate([r[:, dy:], r[:, :dy]], axis=1)
            if dx:
                r = jnp.concatenate([r[:, :, dx:], r[:, :, :dx]], axis=2)
            pieces.append(r)
    p = jnp.concatenate(pieces, axis=-1).reshape(m, 216)
    acc_ref[...] = jnp.dot(p, wq_ref[...],
                           preferred_element_type=jnp.float32)
    bias = b_ref[...]
    a = jnp.maximum(acc_ref[...].reshape(bgrp, 16, 2, 32, 512) + bias, 0.0)
    hmax = jnp.maximum(a[:, :, 0], a[:, :, 1])
    hr = hmax.reshape(bgrp, 16, 16, 1024)
    pooled = jnp.maximum(hr[..., 0:512], hr[..., 512:1024])
    hp = jax.lax.broadcasted_iota(jnp.int32, pooled.shape, 1)
    wp = jax.lax.broadcasted_iota(jnp.int32, pooled.shape, 2)
    interior = ((hp >= 1) & (hp <= 14) & (wp >= 1) & (wp <= 14))
    o_ref[...] = jnp.where(interior, pooled, 0.0).astype(o_ref.dtype)


def _conv1(x, w1, b1, *, bgrp=4):
    n = x.shape[0]
    g = n // 8
    xg = x.reshape(g, 8, 3, 28, 28).transpose(0, 3, 4, 1, 2)
    xg = xg.reshape(g, 28, 28, 24).astype(jnp.bfloat16)
    xg = jnp.pad(xg, ((0, 0), (3, 1), (3, 1), (0, 0)))
    w9 = w1.reshape(9, 1, 3, 1, 64).astype(jnp.bfloat16)
    wq = (jnp.eye(8, dtype=jnp.bfloat16)[None, :, None, :, None]
          * w9).reshape(216, 512)
    bt = jnp.tile(b1, 8)
    yg = pl.pallas_call(
        partial(_conv1_kernel, bgrp=bgrp),
        out_shape=jax.ShapeDtypeStruct((g, 16, 16, 512), jnp.bfloat16),
        grid=(g // bgrp,),
        in_specs=[
            pl.BlockSpec((bgrp, 32, 32, 24), lambda i: (i, 0, 0, 0)),
            pl.BlockSpec((216, 512), lambda i: (0, 0)),
            pl.BlockSpec((1, 512), lambda i: (0, 0)),
        ],
        out_specs=pl.BlockSpec((bgrp, 16, 16, 512), lambda i: (i, 0, 0, 0)),
        scratch_shapes=[pltpu.VMEM((bgrp * 1024, 512), jnp.float32)],
        compiler_params=pltpu.CompilerParams(
            dimension_semantics=("parallel",)),
    )(xg, wq, bt.reshape(1, 512))
    return yg


def _conv2_kernel(x_ref, w_ref, b_ref, o_ref, p_ref, acc_ref, *, bg):
    x = x_ref[...]
    m1 = bg * 224
    for i in range(8):
        xi = x[..., i * 64:(i + 1) * 64]
        for dy in range(3):
            xs = xi[:, dy: dy + 14]
            for dx in range(3):
                t = dy * 3 + dx
                r = xs
                if dx:
                    r = jnp.concatenate([r[:, :, dx:], r[:, :, :dx]], axis=2)
                p_ref[pl.ds(i * m1, m1), pl.ds(t * 64, 64)] = (
                    r.reshape(m1, 64))
    acc_ref[...] = jnp.dot(p_ref[...], w_ref[...].reshape(576, 512),
                           preferred_element_type=jnp.float32)
    bias = b_ref[...]
    for i in range(8):
        a = acc_ref[pl.ds(i * m1, m1), :]
        z = jnp.maximum(a + bias, 0.0).reshape(bg, 7, 2, 16, 512)
        hs = z[:, :, 0] + z[:, :, 1]
        hr = hs.reshape(bg, 7, 8, 1024)
        pooled = (hr[..., 0:512] + hr[..., 512:1024]) * 0.25
        o_ref[:, i] = pooled.astype(o_ref.dtype)


def _conv2(yg, w2, b2, *, bg=2):
    g = yg.shape[0]
    wk = w2.reshape(9, 64, 512).astype(jnp.bfloat16)
    return pl.pallas_call(
        partial(_conv2_kernel, bg=bg),
        out_shape=jax.ShapeDtypeStruct((g, 8, 7, 8, 512), jnp.bfloat16),
        grid=(g // bg,),
        in_specs=[
            pl.BlockSpec((bg, 16, 16, 512), lambda s: (s, 0, 0, 0)),
            pl.BlockSpec((9, 64, 512), lambda s: (0, 0, 0)),
            pl.BlockSpec((1, 512), lambda s: (0, 0)),
        ],
        out_specs=pl.BlockSpec((bg, 8, 7, 8, 512), lambda s: (s, 0, 0, 0, 0)),
        scratch_shapes=[pltpu.VMEM((8 * bg * 224, 576), jnp.bfloat16),
                        pltpu.VMEM((8 * bg * 224, 512), jnp.float32)],
        compiler_params=pltpu.CompilerParams(
            dimension_semantics=("parallel",)),
    )(yg, wk, b2.reshape(1, 512))


def _fc1_kernel(x_ref, w_ref, b_ref, o_ref, acc_ref):
    k = pl.program_id(1)

    @pl.when(k == 0)
    def _():
        acc_ref[...] = jnp.zeros_like(acc_ref)

    acc_ref[...] += jnp.dot(x_ref[...], w_ref[...],
                            preferred_element_type=jnp.float32)

    @pl.when(k == pl.num_programs(1) - 1)
    def _():
        o_ref[...] = jnp.maximum(acc_ref[...] + b_ref[...],
                                 0.0).astype(o_ref.dtype)


def _fc1(x, w, b, *, tn=2048, tk=3584):
    bsz, kdim = x.shape
    ndim = w.shape[1]
    return pl.pallas_call(
        _fc1_kernel,
        out_shape=jax.ShapeDtypeStruct((bsz, ndim), jnp.bfloat16),
        grid=(ndim // tn, kdim // tk),
        in_specs=[
            pl.BlockSpec((bsz, tk), lambda j, k: (0, k)),
            pl.BlockSpec((tk, tn), lambda j, k: (k, j)),
            pl.BlockSpec((1, tn), lambda j, k: (0, j)),
        ],
        out_specs=pl.BlockSpec((bsz, tn), lambda j, k: (0, j)),
        scratch_shapes=[pltpu.VMEM((bsz, tn), jnp.float32)],
        compiler_params=pltpu.CompilerParams(
            dimension_semantics=("parallel", "arbitrary"),
            vmem_limit_bytes=48 * 1024 * 1024),
        cost_estimate=pl.CostEstimate(
            flops=2 * bsz * kdim * ndim, transcendentals=0,
            bytes_accessed=kdim * ndim * 2 + bsz * kdim * 2 + bsz * ndim * 2),
    )(x, w, b.reshape(1, ndim))


def _fc23_kernel(x_ref, w2_ref, b2_ref, w3_ref, b3_ref, o_ref, h_ref):
    h = jnp.dot(x_ref[...], w2_ref[...], preferred_element_type=jnp.float32)
    h_ref[...] = jnp.maximum(h + b2_ref[...], 0.0).astype(jnp.bfloat16)
    y = jnp.dot(h_ref[...], w3_ref[...], preferred_element_type=jnp.float32)
    o_ref[...] = (y + b3_ref[...])[:, :10]


def _fc23(x, w2, b2, w3, b3, *, bm=128):
    bsz, kdim = x.shape
    n2 = w2.shape[1]
    return pl.pallas_call(
        _fc23_kernel,
        out_shape=jax.ShapeDtypeStruct((bsz, 10), jnp.float32),
        grid=(bsz // bm,),
        in_specs=[
            pl.BlockSpec((bm, kdim), lambda i: (i, 0)),
            pl.BlockSpec((kdim, n2), lambda i: (0, 0)),
            pl.BlockSpec((1, n2), lambda i: (0, 0)),
            pl.BlockSpec((n2, w3.shape[1]), lambda i: (0, 0)),
            pl.BlockSpec((1, w3.shape[1]), lambda i: (0, 0)),
        ],
        out_specs=pl.BlockSpec((bm, 10), lambda i: (i, 0)),
        scratch_shapes=[pltpu.VMEM((bm, n2), jnp.bfloat16)],
        compiler_params=pltpu.CompilerParams(
            dimension_semantics=("parallel",),
            vmem_limit_bytes=48 * 1024 * 1024),
        cost_estimate=pl.CostEstimate(
            flops=2 * bsz * kdim * (n2 + w3.shape[1]), transcendentals=0,
            bytes_accessed=kdim * n2 * 2 + bsz * kdim * 2),
    )(x, w2, b2.reshape(1, n2), w3, b3.reshape(1, w3.shape[1]))


def kernel(x, conv1_w, conv1_b, conv2_w, conv2_b,
           fc1_w, fc1_b, fc2_w, fc2_b, fc3_w, fc3_b):
    n = x.shape[0]
    g = n // 8
    yg = _conv1(x, conv1_w, conv1_b, bgrp=min(4, g))
    y2 = _conv2(yg, conv2_w, conv2_b, bg=min(2, g))
    flat = jnp.transpose(y2[:, :, :, :7, :],
                         (0, 1, 4, 2, 3)).reshape(n, 512 * 49)
    h1 = _fc1(flat, fc1_w, fc1_b)
    return _fc23(h1, fc2_w, fc2_b, fc3_w, fc3_b,
                 bm=min(128, n))
